# baseline (speedup 1.0000x reference)
import jax
import jax.numpy as jnp
from jax import lax
from jax.experimental import pallas as pl
from jax.experimental.pallas import tpu as pltpu
from jax.experimental.pallas import tpu_sc as plsc

N = 10000
NP = 10240
E = 320000
NCORE = 2
NSUB = 16
NW = NCORE * NSUB
EPW = E // NW
CHUNK = 500
NCHUNK = EPW // CHUNK
RPT = NP // NSUB

BLK = 400
GRID = N // BLK


def _mesh():
    return plsc.VectorSubcoreMesh(core_axis_name="c", subcore_axis_name="s")


def _deg_body(dst3, zeros16, ones16, out, dst_v, ones_v, dsem, acc):
    c = lax.axis_index("c")
    s = lax.axis_index("s")
    wid = s * NCORE + c
    pltpu.sync_copy(dst3.at[wid], dst_v)
    pltpu.sync_copy(ones16, ones_v)
    r0 = s * RPT
    pltpu.sync_copy(zeros16.at[pl.ds(r0, RPT)], acc.at[pl.ds(r0, RPT)])
    plsc.subcore_barrier()

    def fire(i, carry):
        for b in range(2):
            j = 2 * i + b
            pltpu.async_copy(ones_v, acc.at[dst_v.at[j]], dsem, add=True)
        return carry

    def drain(i, carry):
        for b in range(2):
            j = 2 * i + b
            pltpu.make_async_copy(ones_v, acc.at[dst_v.at[j]], dsem).wait()
        return carry

    lax.fori_loop(0, NCHUNK // 2, fire, 0)
    lax.fori_loop(0, NCHUNK // 2, drain, 0)
    plsc.subcore_barrier()
    pltpu.sync_copy(acc.at[pl.ds(r0, RPT)], out.at[c, pl.ds(r0, RPT)])


def _sc_deg(dst3, zeros16, ones16):
    return pl.kernel(
        _deg_body,
        mesh=_mesh(),
        compiler_params=pltpu.CompilerParams(use_tc_tiling_on_sc=False),
        out_type=jax.ShapeDtypeStruct((NCORE, NP, 16), jnp.float32),
        scratch_types=[
            pltpu.VMEM((NCHUNK, CHUNK), jnp.int32),
            pltpu.VMEM((CHUNK, 16), jnp.float32),
            pltpu.SemaphoreType.DMA,
            pltpu.VMEM_SHARED((NP, 16), jnp.float32),
        ],
    )(dst3, zeros16, ones16)


def _spmm_body(vrows, src3, dst3, zeros, out,
               src_v, dst_v, buf0, buf1, gsem0, gsem1, ssem0, ssem1, acc):
    c = lax.axis_index("c")
    s = lax.axis_index("s")
    wid = s * NCORE + c
    pltpu.sync_copy(src3.at[wid], src_v)
    pltpu.sync_copy(dst3.at[wid], dst_v)
    r0 = s * RPT
    pltpu.sync_copy(zeros.at[pl.ds(r0, RPT)], acc.at[pl.ds(r0, RPT)])
    plsc.subcore_barrier()

    bufs = (buf0, buf1)
    gsems = (gsem0, gsem1)
    ssems = (ssem0, ssem1)
    pltpu.async_copy(vrows.at[src_v.at[0]], buf0, gsem0)

    def body(i, carry):
        for b in range(2):
            j = 2 * i + b
            nxt = 1 - b

            @pl.when(j >= 1)
            def _():
                pltpu.make_async_copy(bufs[nxt], acc.at[dst_v.at[j - 1]],
                                      ssems[nxt]).wait()

            @pl.when(j + 1 < NCHUNK)
            def _():
                pltpu.async_copy(vrows.at[src_v.at[j + 1]], bufs[nxt], gsems[nxt])

            pltpu.make_async_copy(vrows.at[src_v.at[j]], bufs[b], gsems[b]).wait()
            pltpu.async_copy(bufs[b], acc.at[dst_v.at[j]], ssems[b], add=True)
        return carry

    lax.fori_loop(0, NCHUNK // 2, body, 0)
    pltpu.make_async_copy(bufs[(NCHUNK - 1) % 2], acc.at[dst_v.at[NCHUNK - 1]],
                          ssems[(NCHUNK - 1) % 2]).wait()
    plsc.subcore_barrier()
    pltpu.sync_copy(acc.at[pl.ds(r0, RPT)], out.at[c, pl.ds(r0, RPT)])


def _sc_spmm(vrows, src3, dst3, zeros):
    return pl.kernel(
        _spmm_body,
        mesh=_mesh(),
        compiler_params=pltpu.CompilerParams(use_tc_tiling_on_sc=False),
        out_type=jax.ShapeDtypeStruct((NCORE, NP, 64), jnp.float32),
        scratch_types=[
            pltpu.VMEM((NCHUNK, CHUNK), jnp.int32),
            pltpu.VMEM((NCHUNK, CHUNK), jnp.int32),
            pltpu.VMEM((CHUNK, 64), jnp.float32),
            pltpu.VMEM((CHUNK, 64), jnp.float32),
            pltpu.SemaphoreType.DMA,
            pltpu.SemaphoreType.DMA,
            pltpu.SemaphoreType.DMA,
            pltpu.SemaphoreType.DMA,
            pltpu.VMEM_SHARED((NP, 64), jnp.float32),
        ],
    )(vrows, src3, dst3, zeros)


def _dinv(degp):
    return lax.rsqrt(degp[0, :, 0] + degp[1, :, 0] + 1.0)


def _headmm_body(x_ref, nsum_ref, w1_ref, y_ref):
    xa = x_ref[...] + nsum_ref[...]
    y_ref[...] = jnp.dot(xa, w1_ref[...], preferred_element_type=jnp.float32)


def _tc_headmm(x, nsum, W1):
    return pl.pallas_call(
        _headmm_body,
        grid=(GRID,),
        in_specs=[
            pl.BlockSpec((BLK, 128), lambda i: (i, 0)),
            pl.BlockSpec((BLK, 128), lambda i: (i, 0)),
            pl.BlockSpec((128, 64), lambda i: (0, 0)),
        ],
        out_specs=pl.BlockSpec((BLK, 64), lambda i: (i, 0)),
        out_shape=jax.ShapeDtypeStruct((NP, 64), jnp.float32),
    )(x, nsum, W1)


def _scale_body(y_ref, degp_ref, yp_ref):
    di = _dinv(degp_ref[...])
    yp_ref[...] = y_ref[...] * di[:, None]


def _tc_scale(y, degp):
    return pl.pallas_call(
        _scale_body,
        grid=(GRID,),
        in_specs=[
            pl.BlockSpec((BLK, 64), lambda i: (i, 0)),
            pl.BlockSpec((NCORE, BLK, 16), lambda i: (0, i, 0)),
        ],
        out_specs=pl.BlockSpec((BLK, 64), lambda i: (i, 0)),
        out_shape=jax.ShapeDtypeStruct((NP, 64), jnp.float32),
    )(y, degp)


def _mid_body(acc_ref, yp_ref, degp_ref, b1_ref, x1_ref, u1_ref):
    a = acc_ref[...]
    di = _dinv(degp_ref[...])
    z1 = (a[0] + a[1] + yp_ref[...]) * di[:, None]
    x1 = z1 + b1_ref[...]
    x1_ref[...] = x1
    u1_ref[...] = x1 * di[:, None]


def _tc_mid(acc1, yp, degp, b1r):
    return pl.pallas_call(
        _mid_body,
        grid=(GRID,),
        in_specs=[
            pl.BlockSpec((NCORE, BLK, 64), lambda i: (0, i, 0)),
            pl.BlockSpec((BLK, 64), lambda i: (i, 0)),
            pl.BlockSpec((NCORE, BLK, 16), lambda i: (0, i, 0)),
            pl.BlockSpec((1, 64), lambda i: (0, 0)),
        ],
        out_specs=[
            pl.BlockSpec((BLK, 64), lambda i: (i, 0)),
            pl.BlockSpec((BLK, 64), lambda i: (i, 0)),
        ],
        out_shape=[
            jax.ShapeDtypeStruct((NP, 64), jnp.float32),
            jax.ShapeDtypeStruct((NP, 64), jnp.float32),
        ],
    )(acc1, yp, degp, b1r)


def _tail_body(acc_ref, u1_ref, degp_ref, x1_ref, wlt_ref, w2_ref, wlb_ref,
               blr_ref, b2r_ref, w3_ref, b3r_ref, w4_ref, b4r_ref,
               hp_ref, ch_ref):
    a = acc_ref[...]
    di = _dinv(degp_ref[...])
    z2 = (a[0] + a[1] + u1_ref[...]) * di[:, None]
    wlb = wlb_ref[...]
    wc = jnp.dot(w2_ref[...], wlb, preferred_element_type=jnp.float32)
    bc = blr_ref[...] + jnp.dot(b2r_ref[...], wlb,
                                preferred_element_type=jnp.float32)
    hp = (jnp.dot(x1_ref[...], wlt_ref[...], preferred_element_type=jnp.float32)
          + jnp.dot(z2, wc, preferred_element_type=jnp.float32) + bc)
    hp_ref[...] = hp
    t = jnp.maximum(jnp.dot(hp, w3_ref[...],
                            preferred_element_type=jnp.float32) + b3r_ref[...], 0.0)
    ch_ref[...] = jnp.dot(t, w4_ref[...],
                          preferred_element_type=jnp.float32) + b4r_ref[...]


def _tc_tail(acc2, u1, degp, x1, Wl_top, W2, Wl_bot, blr, b2r, W3, b3r, W4, b4r):
    return pl.pallas_call(
        _tail_body,
        grid=(GRID,),
        in_specs=[
            pl.BlockSpec((NCORE, BLK, 64), lambda i: (0, i, 0)),
            pl.BlockSpec((BLK, 64), lambda i: (i, 0)),
            pl.BlockSpec((NCORE, BLK, 16), lambda i: (0, i, 0)),
            pl.BlockSpec((BLK, 64), lambda i: (i, 0)),
            pl.BlockSpec((64, 128), lambda i: (0, 0)),
            pl.BlockSpec((64, 128), lambda i: (0, 0)),
            pl.BlockSpec((128, 128), lambda i: (0, 0)),
            pl.BlockSpec((1, 128), lambda i: (0, 0)),
            pl.BlockSpec((1, 128), lambda i: (0, 0)),
            pl.BlockSpec((128, 256), lambda i: (0, 0)),
            pl.BlockSpec((1, 256), lambda i: (0, 0)),
            pl.BlockSpec((256, 128), lambda i: (0, 0)),
            pl.BlockSpec((1, 128), lambda i: (0, 0)),
        ],
        out_specs=[
            pl.BlockSpec((BLK, 128), lambda i: (i, 0)),
            pl.BlockSpec((BLK, 128), lambda i: (i, 0)),
        ],
        out_shape=[
            jax.ShapeDtypeStruct((N, 128), jnp.float32),
            jax.ShapeDtypeStruct((N, 128), jnp.float32),
        ],
    )(acc2, u1, degp, x1, Wl_top, W2, Wl_bot, blr, b2r, W3, b3r, W4, b4r)


def _tf2x32(k0, k1, x0, x1):
    ks0 = _np.uint32(k0); ks1 = _np.uint32(k1)
    ks2 = _np.uint32(_np.uint32(0x1BD11BDA) ^ ks0 ^ ks1)
    ks = (ks0, ks1, ks2)
    x0 = (x0 + ks0).astype(_np.uint32); x1 = (x1 + ks1).astype(_np.uint32)
    rots = ((13, 15, 26, 6), (17, 29, 16, 24))
    for i in range(5):
        for r in rots[i % 2]:
            x0 = (x0 + x1).astype(_np.uint32)
            x1 = ((x1 << _np.uint32(r)) | (x1 >> _np.uint32(32 - r))).astype(_np.uint32)
            x1 = (x1 ^ x0).astype(_np.uint32)
        x0 = (x0 + ks[(i + 1) % 3]).astype(_np.uint32)
        x1 = (x1 + ks[(i + 2) % 3] + _np.uint32(i + 1)).astype(_np.uint32)
    return x0, x1


def _ndtri(p):
    a = [-3.969683028665376e+01, 2.209460984245205e+02, -2.759285104469687e+02,
         1.383577518672690e+02, -3.066479806614716e+01, 2.506628277459239e+00]
    b = [-5.447609879822406e+01, 1.615858368580409e+02, -1.556989798598866e+02,
         6.680131188771972e+01, -1.328068155288572e+01]
    c = [-7.784894002430293e-03, -3.223964580411365e-01, -2.400758277161838e+00,
         -2.549732539343734e+00, 4.374664141464968e+00, 2.938163982698783e+00]
    d = [7.784695709041462e-03, 3.224671290700398e-01, 2.445134137142996e+00,
         3.754408661907416e+00]
    p = _np.asarray(p, _np.float64)
    out = _np.empty_like(p)
    plow = 0.02425
    lo = p < plow; hi = p > 1 - plow; mid = ~(lo | hi)
    q = _np.sqrt(-2 * _np.log(p[lo]))
    out[lo] = (((((c[0]*q+c[1])*q+c[2])*q+c[3])*q+c[4])*q+c[5]) / ((((d[0]*q+d[1])*q+d[2])*q+d[3])*q+1)
    q = _np.sqrt(-2 * _np.log(1 - p[hi]))
    out[hi] = -(((((c[0]*q+c[1])*q+c[2])*q+c[3])*q+c[4])*q+c[5]) / ((((d[0]*q+d[1])*q+d[2])*q+d[3])*q+1)
    q = p[mid] - 0.5; r = q * q
    out[mid] = (((((a[0]*r+a[1])*r+a[2])*r+a[3])*r+a[4])*r+a[5])*q / (((((b[0]*r+b[1])*r+b[2])*r+b[3])*r+b[4])*r+1)
    return out


def _np_normal(k0, k1, n):
    cnt = _np.arange(n, dtype=_np.uint32)
    a, b = _tf2x32(k0, k1, _np.zeros(n, _np.uint32), cnt)
    bits = a ^ b
    f = ((bits >> _np.uint32(9)) | _np.uint32(0x3F800000)).view(_np.float32)
    u01 = (f - _np.float32(1.0)).astype(_np.float32)
    lo = _np.float32(_np.nextafter(_np.float32(-1.0), _np.float32(0.0)))
    hi = _np.float32(1.0)
    u = _np.maximum(lo, (u01 * (hi - lo) + lo).astype(_np.float32))
    return _ndtri((u.astype(_np.float64) + 1.0) / 2.0).astype(_np.float32)


def _const_noise():
    def fold(d):
        a, b = _tf2x32(_np.uint32(0), _np.uint32(42),
                       _np.uint32([0]), _np.uint32([d]))
        return a[0], b[0]
    k1 = fold(1)
    k2 = fold(2)
    n1 = _np_normal(k1[0], k1[1], N * 128).reshape(N, 128)
    n2 = _np_normal(k2[0], k2[1], N * 128).reshape(N, 128)
    return ((n1 + n2) * _np.float32(0.1)).astype(_np.float32)


_np = __import__("numpy")
_NSUM = _const_noise()
_ZEROS = _np.zeros((NP, 64), "float32")
_ZEROS16 = _np.zeros((NP, 16), "float32")
_ONES16 = _np.ones((CHUNK, 16), "float32")


def kernel(x, edge_index, W1, b1, W2, b2, Wl, bl, W3, b3, W4, b4):
    nsum = jnp.asarray(_NSUM)

    src3 = edge_index[0].reshape(NW, NCHUNK, CHUNK)
    dst3 = edge_index[1].reshape(NW, NCHUNK, CHUNK)
    zeros = jnp.asarray(_ZEROS)
    zeros16 = jnp.asarray(_ZEROS16)
    ones16 = jnp.asarray(_ONES16)

    degp = _sc_deg(dst3, zeros16, ones16)
    y = _tc_headmm(x, nsum, W1)
    yp = _tc_scale(y, degp)
    acc1 = _sc_spmm(yp, src3, dst3, zeros)
    x1, u1 = _tc_mid(acc1, yp, degp, b1.reshape(1, 64))
    acc2 = _sc_spmm(u1, src3, dst3, zeros)
    hp, ch = _tc_tail(acc2, u1, degp, x1, Wl[:64], W2, Wl[64:],
                      bl.reshape(1, 128), b2.reshape(1, 128),
                      W3, b3.reshape(1, 256), W4, b4.reshape(1, 128))
    h = hp[None]
    c = ch[None]
    return (h, h, c, c)


# speedup vs baseline: 1.0195x; 1.0195x over previous
import jax
import jax.numpy as jnp
from jax import lax
from jax.experimental import pallas as pl
from jax.experimental.pallas import tpu as pltpu
from jax.experimental.pallas import tpu_sc as plsc

N = 10000
NP = 10240
E = 320000
NCORE = 2
NSUB = 16
NW = NCORE * NSUB
EPW = E // NW
CHUNK = 500
NCHUNK = EPW // CHUNK
RPT = NP // NSUB

BLK = 400
GRID = N // BLK


def _mesh():
    return plsc.VectorSubcoreMesh(core_axis_name="c", subcore_axis_name="s")


NROW = NP // 16
RROW = NROW // NSUB


def _deg_body(dst2, zeros16, id2, out, dstf, idv, degt, rsem, acc):
    c = lax.axis_index("c")
    s = lax.axis_index("s")
    wid = s * NCORE + c
    pltpu.sync_copy(dst2.at[wid], dstf)
    pltpu.sync_copy(id2, idv)
    pltpu.sync_copy(zeros16.at[pl.ds(0, NROW)], degt)
    r0 = s * RROW
    pltpu.sync_copy(zeros16.at[pl.ds(r0, RROW)], acc.at[pl.ds(r0, RROW)])

    ones = jnp.ones((16,), jnp.float32)

    def count(i, carry):
        idx = dstf[pl.ds(i * 16, 16)]
        plsc.addupdate_scatter(degt, [idx >> 4, idx & 15], ones)
        return carry

    lax.fori_loop(0, EPW // 16, count, 0)
    plsc.subcore_barrier()
    for g in range(NROW // 128):
        pltpu.async_copy(degt.at[pl.ds(g * 128, 128)], acc.at[idv.at[g]],
                         rsem, add=True)
    for g in range(NROW // 128):
        pltpu.make_async_copy(degt.at[pl.ds(g * 128, 128)], acc.at[idv.at[g]],
                              rsem).wait()
    plsc.subcore_barrier()
    pltpu.sync_copy(acc.at[pl.ds(r0, RROW)], out.at[c, pl.ds(r0, RROW)])


def _sc_deg(dst2, zeros16, id2):
    return pl.kernel(
        _deg_body,
        mesh=_mesh(),
        compiler_params=pltpu.CompilerParams(use_tc_tiling_on_sc=False,
                                             needs_layout_passes=False),
        out_type=jax.ShapeDtypeStruct((NCORE, NROW, 16), jnp.float32),
        scratch_types=[
            pltpu.VMEM((EPW,), jnp.int32),
            pltpu.VMEM((NROW // 128, 128), jnp.int32),
            pltpu.VMEM((NROW, 16), jnp.float32),
            pltpu.SemaphoreType.DMA,
            pltpu.VMEM_SHARED((NROW, 16), jnp.float32),
        ],
    )(dst2, zeros16, id2)


def _spmm_body(vrows, src3, dst3, zeros, out,
               src_v, dst_v, buf0, buf1, gsem0, gsem1, ssem0, ssem1, acc):
    c = lax.axis_index("c")
    s = lax.axis_index("s")
    wid = s * NCORE + c
    pltpu.sync_copy(src3.at[wid], src_v)
    pltpu.sync_copy(dst3.at[wid], dst_v)
    r0 = s * RPT
    pltpu.sync_copy(zeros.at[pl.ds(r0, RPT)], acc.at[pl.ds(r0, RPT)])
    plsc.subcore_barrier()

    bufs = (buf0, buf1)
    gsems = (gsem0, gsem1)
    ssems = (ssem0, ssem1)
    pltpu.async_copy(vrows.at[src_v.at[0]], buf0, gsem0)

    def body(i, carry):
        for b in range(2):
            j = 2 * i + b
            nxt = 1 - b

            @pl.when(j >= 1)
            def _():
                pltpu.make_async_copy(bufs[nxt], acc.at[dst_v.at[j - 1]],
                                      ssems[nxt]).wait()

            @pl.when(j + 1 < NCHUNK)
            def _():
                pltpu.async_copy(vrows.at[src_v.at[j + 1]], bufs[nxt], gsems[nxt])

            pltpu.make_async_copy(vrows.at[src_v.at[j]], bufs[b], gsems[b]).wait()
            pltpu.async_copy(bufs[b], acc.at[dst_v.at[j]], ssems[b], add=True)
        return carry

    lax.fori_loop(0, NCHUNK // 2, body, 0)
    pltpu.make_async_copy(bufs[(NCHUNK - 1) % 2], acc.at[dst_v.at[NCHUNK - 1]],
                          ssems[(NCHUNK - 1) % 2]).wait()
    plsc.subcore_barrier()
    pltpu.sync_copy(acc.at[pl.ds(r0, RPT)], out.at[c, pl.ds(r0, RPT)])


def _sc_spmm(vrows, src3, dst3, zeros):
    return pl.kernel(
        _spmm_body,
        mesh=_mesh(),
        compiler_params=pltpu.CompilerParams(use_tc_tiling_on_sc=False),
        out_type=jax.ShapeDtypeStruct((NCORE, NP, 64), jnp.float32),
        scratch_types=[
            pltpu.VMEM((NCHUNK, CHUNK), jnp.int32),
            pltpu.VMEM((NCHUNK, CHUNK), jnp.int32),
            pltpu.VMEM((CHUNK, 64), jnp.float32),
            pltpu.VMEM((CHUNK, 64), jnp.float32),
            pltpu.SemaphoreType.DMA,
            pltpu.SemaphoreType.DMA,
            pltpu.SemaphoreType.DMA,
            pltpu.SemaphoreType.DMA,
            pltpu.VMEM_SHARED((NP, 64), jnp.float32),
        ],
    )(vrows, src3, dst3, zeros)


def _dinv(degp):
    return lax.rsqrt(degp[0, :, 0] + degp[1, :, 0] + 1.0)


def _head_body(x_ref, nsum_ref, w1_ref, degp_ref, yp_ref):
    xa = x_ref[...] + nsum_ref[...]
    di = _dinv(degp_ref[...])
    y = jnp.dot(xa, w1_ref[...], preferred_element_type=jnp.float32)
    yp_ref[...] = y * di[:, None]


def _tc_head(x, nsum, W1, degp):
    return pl.pallas_call(
        _head_body,
        grid=(GRID,),
        in_specs=[
            pl.BlockSpec((BLK, 128), lambda i: (i, 0)),
            pl.BlockSpec((BLK, 128), lambda i: (i, 0)),
            pl.BlockSpec((128, 64), lambda i: (0, 0)),
            pl.BlockSpec((NCORE, BLK, 1), lambda i: (0, i, 0)),
        ],
        out_specs=pl.BlockSpec((BLK, 64), lambda i: (i, 0)),
        out_shape=jax.ShapeDtypeStruct((NP, 64), jnp.float32),
    )(x, nsum, W1, degp)


def _mid_body(acc_ref, yp_ref, degp_ref, b1_ref, x1_ref, u1_ref):
    a = acc_ref[...]
    di = _dinv(degp_ref[...])
    z1 = (a[0] + a[1] + yp_ref[...]) * di[:, None]
    x1 = z1 + b1_ref[...]
    x1_ref[...] = x1
    u1_ref[...] = x1 * di[:, None]


def _tc_mid(acc1, yp, degp, b1r):
    return pl.pallas_call(
        _mid_body,
        grid=(GRID,),
        in_specs=[
            pl.BlockSpec((NCORE, BLK, 64), lambda i: (0, i, 0)),
            pl.BlockSpec((BLK, 64), lambda i: (i, 0)),
            pl.BlockSpec((NCORE, BLK, 1), lambda i: (0, i, 0)),
            pl.BlockSpec((1, 64), lambda i: (0, 0)),
        ],
        out_specs=[
            pl.BlockSpec((BLK, 64), lambda i: (i, 0)),
            pl.BlockSpec((BLK, 64), lambda i: (i, 0)),
        ],
        out_shape=[
            jax.ShapeDtypeStruct((NP, 64), jnp.float32),
            jax.ShapeDtypeStruct((NP, 64), jnp.float32),
        ],
    )(acc1, yp, degp, b1r)


def _tail_body(acc_ref, u1_ref, degp_ref, x1_ref, wlt_ref, w2_ref, wlb_ref,
               blr_ref, b2r_ref, w3_ref, b3r_ref, w4_ref, b4r_ref,
               hp_ref, ch_ref):
    a = acc_ref[...]
    di = _dinv(degp_ref[...])
    z2 = (a[0] + a[1] + u1_ref[...]) * di[:, None]
    wlb = wlb_ref[...]
    wc = jnp.dot(w2_ref[...], wlb, preferred_element_type=jnp.float32)
    bc = blr_ref[...] + jnp.dot(b2r_ref[...], wlb,
                                preferred_element_type=jnp.float32)
    hp = (jnp.dot(x1_ref[...], wlt_ref[...], preferred_element_type=jnp.float32)
          + jnp.dot(z2, wc, preferred_element_type=jnp.float32) + bc)
    hp_ref[...] = hp
    t = jnp.maximum(jnp.dot(hp, w3_ref[...],
                            preferred_element_type=jnp.float32) + b3r_ref[...], 0.0)
    ch_ref[...] = jnp.dot(t, w4_ref[...],
                          preferred_element_type=jnp.float32) + b4r_ref[...]


def _tc_tail(acc2, u1, degp, x1, Wl_top, W2, Wl_bot, blr, b2r, W3, b3r, W4, b4r):
    return pl.pallas_call(
        _tail_body,
        grid=(GRID,),
        in_specs=[
            pl.BlockSpec((NCORE, BLK, 64), lambda i: (0, i, 0)),
            pl.BlockSpec((BLK, 64), lambda i: (i, 0)),
            pl.BlockSpec((NCORE, BLK, 1), lambda i: (0, i, 0)),
            pl.BlockSpec((BLK, 64), lambda i: (i, 0)),
            pl.BlockSpec((64, 128), lambda i: (0, 0)),
            pl.BlockSpec((64, 128), lambda i: (0, 0)),
            pl.BlockSpec((128, 128), lambda i: (0, 0)),
            pl.BlockSpec((1, 128), lambda i: (0, 0)),
            pl.BlockSpec((1, 128), lambda i: (0, 0)),
            pl.BlockSpec((128, 256), lambda i: (0, 0)),
            pl.BlockSpec((1, 256), lambda i: (0, 0)),
            pl.BlockSpec((256, 128), lambda i: (0, 0)),
            pl.BlockSpec((1, 128), lambda i: (0, 0)),
        ],
        out_specs=[
            pl.BlockSpec((BLK, 128), lambda i: (i, 0)),
            pl.BlockSpec((BLK, 128), lambda i: (i, 0)),
        ],
        out_shape=[
            jax.ShapeDtypeStruct((N, 128), jnp.float32),
            jax.ShapeDtypeStruct((N, 128), jnp.float32),
        ],
    )(acc2, u1, degp, x1, Wl_top, W2, Wl_bot, blr, b2r, W3, b3r, W4, b4r)


def _tf2x32(k0, k1, x0, x1):
    ks0 = _np.uint32(k0); ks1 = _np.uint32(k1)
    ks2 = _np.uint32(_np.uint32(0x1BD11BDA) ^ ks0 ^ ks1)
    ks = (ks0, ks1, ks2)
    x0 = (x0 + ks0).astype(_np.uint32); x1 = (x1 + ks1).astype(_np.uint32)
    rots = ((13, 15, 26, 6), (17, 29, 16, 24))
    for i in range(5):
        for r in rots[i % 2]:
            x0 = (x0 + x1).astype(_np.uint32)
            x1 = ((x1 << _np.uint32(r)) | (x1 >> _np.uint32(32 - r))).astype(_np.uint32)
            x1 = (x1 ^ x0).astype(_np.uint32)
        x0 = (x0 + ks[(i + 1) % 3]).astype(_np.uint32)
        x1 = (x1 + ks[(i + 2) % 3] + _np.uint32(i + 1)).astype(_np.uint32)
    return x0, x1


def _ndtri(p):
    a = [-3.969683028665376e+01, 2.209460984245205e+02, -2.759285104469687e+02,
         1.383577518672690e+02, -3.066479806614716e+01, 2.506628277459239e+00]
    b = [-5.447609879822406e+01, 1.615858368580409e+02, -1.556989798598866e+02,
         6.680131188771972e+01, -1.328068155288572e+01]
    c = [-7.784894002430293e-03, -3.223964580411365e-01, -2.400758277161838e+00,
         -2.549732539343734e+00, 4.374664141464968e+00, 2.938163982698783e+00]
    d = [7.784695709041462e-03, 3.224671290700398e-01, 2.445134137142996e+00,
         3.754408661907416e+00]
    p = _np.asarray(p, _np.float64)
    out = _np.empty_like(p)
    plow = 0.02425
    lo = p < plow; hi = p > 1 - plow; mid = ~(lo | hi)
    q = _np.sqrt(-2 * _np.log(p[lo]))
    out[lo] = (((((c[0]*q+c[1])*q+c[2])*q+c[3])*q+c[4])*q+c[5]) / ((((d[0]*q+d[1])*q+d[2])*q+d[3])*q+1)
    q = _np.sqrt(-2 * _np.log(1 - p[hi]))
    out[hi] = -(((((c[0]*q+c[1])*q+c[2])*q+c[3])*q+c[4])*q+c[5]) / ((((d[0]*q+d[1])*q+d[2])*q+d[3])*q+1)
    q = p[mid] - 0.5; r = q * q
    out[mid] = (((((a[0]*r+a[1])*r+a[2])*r+a[3])*r+a[4])*r+a[5])*q / (((((b[0]*r+b[1])*r+b[2])*r+b[3])*r+b[4])*r+1)
    return out


def _np_normal(k0, k1, n):
    cnt = _np.arange(n, dtype=_np.uint32)
    a, b = _tf2x32(k0, k1, _np.zeros(n, _np.uint32), cnt)
    bits = a ^ b
    f = ((bits >> _np.uint32(9)) | _np.uint32(0x3F800000)).view(_np.float32)
    u01 = (f - _np.float32(1.0)).astype(_np.float32)
    lo = _np.float32(_np.nextafter(_np.float32(-1.0), _np.float32(0.0)))
    hi = _np.float32(1.0)
    u = _np.maximum(lo, (u01 * (hi - lo) + lo).astype(_np.float32))
    return _ndtri((u.astype(_np.float64) + 1.0) / 2.0).astype(_np.float32)


def _const_noise():
    def fold(d):
        a, b = _tf2x32(_np.uint32(0), _np.uint32(42),
                       _np.uint32([0]), _np.uint32([d]))
        return a[0], b[0]
    k1 = fold(1)
    k2 = fold(2)
    n1 = _np_normal(k1[0], k1[1], N * 128).reshape(N, 128)
    n2 = _np_normal(k2[0], k2[1], N * 128).reshape(N, 128)
    return ((n1 + n2) * _np.float32(0.1)).astype(_np.float32)


_np = __import__("numpy")
_NSUM = _const_noise()
_ZEROS = _np.zeros((NP, 64), "float32")
_ZEROS16 = _np.zeros((NP, 16), "float32")
_ID2 = _np.arange(640, dtype="int32").reshape(5, 128)


def kernel(x, edge_index, W1, b1, W2, b2, Wl, bl, W3, b3, W4, b4):
    nsum = jnp.asarray(_NSUM)

    src3 = edge_index[0].reshape(NW, NCHUNK, CHUNK)
    dst3 = edge_index[1].reshape(NW, NCHUNK, CHUNK)
    zeros = jnp.asarray(_ZEROS)
    zeros16 = jnp.asarray(_ZEROS16)
    id2 = jnp.asarray(_ID2)

    dst2 = edge_index[1].reshape(NW, EPW)
    degp = _sc_deg(dst2, zeros16, id2).reshape(NCORE, NP, 1)
    yp = _tc_head(x, nsum, W1, degp)
    acc1 = _sc_spmm(yp, src3, dst3, zeros)
    x1, u1 = _tc_mid(acc1, yp, degp, b1.reshape(1, 64))
    acc2 = _sc_spmm(u1, src3, dst3, zeros)
    hp, ch = _tc_tail(acc2, u1, degp, x1, Wl[:64], W2, Wl[64:],
                      bl.reshape(1, 128), b2.reshape(1, 128),
                      W3, b3.reshape(1, 256), W4, b4.reshape(1, 128))
    h = hp[None]
    c = ch[None]
    return (h, h, c, c)


# speedup vs baseline: 1.0258x; 1.0062x over previous
import jax
import jax.numpy as jnp
from jax import lax
from jax.experimental import pallas as pl
from jax.experimental.pallas import tpu as pltpu
from jax.experimental.pallas import tpu_sc as plsc

N = 10000
NP = 10240
E = 320000
NCORE = 2
NSUB = 16
NW = NCORE * NSUB
EPW = E // NW
CHUNK = 500
NCHUNK = EPW // CHUNK
RPT = NP // NSUB

BLK = 400
GRID = N // BLK


def _mesh():
    return plsc.VectorSubcoreMesh(core_axis_name="c", subcore_axis_name="s")


def _deg_body(dst3, zeros16, ones16, out, dst_v, ones_v, dsem, acc):
    c = lax.axis_index("c")
    s = lax.axis_index("s")
    wid = s * NCORE + c
    pltpu.sync_copy(dst3.at[wid], dst_v)
    pltpu.sync_copy(ones16, ones_v)
    r0 = s * RPT
    pltpu.sync_copy(zeros16.at[pl.ds(r0, RPT)], acc.at[pl.ds(r0, RPT)])
    plsc.subcore_barrier()

    def fire(i, carry):
        for b in range(2):
            j = 2 * i + b
            pltpu.async_copy(ones_v, acc.at[dst_v.at[j]], dsem, add=True)
        return carry

    def drain(i, carry):
        for b in range(2):
            j = 2 * i + b
            pltpu.make_async_copy(ones_v, acc.at[dst_v.at[j]], dsem).wait()
        return carry

    lax.fori_loop(0, NCHUNK // 2, fire, 0)
    lax.fori_loop(0, NCHUNK // 2, drain, 0)
    plsc.subcore_barrier()
    pltpu.sync_copy(acc.at[pl.ds(r0, RPT)], out.at[c, pl.ds(r0, RPT)])


def _sc_deg(dst3, zeros16, ones16):
    return pl.kernel(
        _deg_body,
        mesh=_mesh(),
        compiler_params=pltpu.CompilerParams(use_tc_tiling_on_sc=False),
        out_type=jax.ShapeDtypeStruct((NCORE, NP, 16), jnp.float32),
        scratch_types=[
            pltpu.VMEM((NCHUNK, CHUNK), jnp.int32),
            pltpu.VMEM((CHUNK, 16), jnp.float32),
            pltpu.SemaphoreType.DMA,
            pltpu.VMEM_SHARED((NP, 16), jnp.float32),
        ],
    )(dst3, zeros16, ones16)


def _spmm_body(vrows, src3, dst3, zeros, out,
               src_v, dst_v, buf0, buf1, gsem0, gsem1, ssem0, ssem1, acc):
    c = lax.axis_index("c")
    s = lax.axis_index("s")
    wid = s * NCORE + c
    pltpu.sync_copy(src3.at[wid], src_v)
    pltpu.sync_copy(dst3.at[wid], dst_v)
    r0 = s * RPT
    pltpu.sync_copy(zeros.at[pl.ds(r0, RPT)], acc.at[pl.ds(r0, RPT)])
    plsc.subcore_barrier()

    bufs = (buf0, buf1)
    gsems = (gsem0, gsem1)
    ssems = (ssem0, ssem1)
    pltpu.async_copy(vrows.at[src_v.at[0]], buf0, gsem0)

    def body(i, carry):
        for b in range(2):
            j = 2 * i + b
            nxt = 1 - b

            @pl.when(j >= 1)
            def _():
                pltpu.make_async_copy(bufs[nxt], acc.at[dst_v.at[j - 1]],
                                      ssems[nxt]).wait()

            @pl.when(j + 1 < NCHUNK)
            def _():
                pltpu.async_copy(vrows.at[src_v.at[j + 1]], bufs[nxt], gsems[nxt])

            pltpu.make_async_copy(vrows.at[src_v.at[j]], bufs[b], gsems[b]).wait()
            pltpu.async_copy(bufs[b], acc.at[dst_v.at[j]], ssems[b], add=True)
        return carry

    lax.fori_loop(0, NCHUNK // 2, body, 0)
    pltpu.make_async_copy(bufs[(NCHUNK - 1) % 2], acc.at[dst_v.at[NCHUNK - 1]],
                          ssems[(NCHUNK - 1) % 2]).wait()
    plsc.subcore_barrier()
    pltpu.sync_copy(acc.at[pl.ds(r0, RPT)], out.at[c, pl.ds(r0, RPT)])


def _sc_spmm(vrows, src3, dst3, zeros):
    return pl.kernel(
        _spmm_body,
        mesh=_mesh(),
        compiler_params=pltpu.CompilerParams(use_tc_tiling_on_sc=False),
        out_type=jax.ShapeDtypeStruct((NCORE, NP, 64), jnp.float32),
        scratch_types=[
            pltpu.VMEM((NCHUNK, CHUNK), jnp.int32),
            pltpu.VMEM((NCHUNK, CHUNK), jnp.int32),
            pltpu.VMEM((CHUNK, 64), jnp.float32),
            pltpu.VMEM((CHUNK, 64), jnp.float32),
            pltpu.SemaphoreType.DMA,
            pltpu.SemaphoreType.DMA,
            pltpu.SemaphoreType.DMA,
            pltpu.SemaphoreType.DMA,
            pltpu.VMEM_SHARED((NP, 64), jnp.float32),
        ],
    )(vrows, src3, dst3, zeros)


def _dinv(degp):
    return lax.rsqrt(degp[0, :, 0] + degp[1, :, 0] + 1.0)


def _head_body(x_ref, nsum_ref, w1_ref, degp_ref, yp_ref):
    xa = x_ref[...] + nsum_ref[...]
    di = _dinv(degp_ref[...])
    y = jnp.dot(xa, w1_ref[...], preferred_element_type=jnp.float32)
    yp_ref[...] = y * di[:, None]


def _tc_head(x, nsum, W1, degp):
    return pl.pallas_call(
        _head_body,
        grid=(GRID,),
        in_specs=[
            pl.BlockSpec((BLK, 128), lambda i: (i, 0)),
            pl.BlockSpec((BLK, 128), lambda i: (i, 0)),
            pl.BlockSpec((128, 64), lambda i: (0, 0)),
            pl.BlockSpec((NCORE, BLK, 16), lambda i: (0, i, 0)),
        ],
        out_specs=pl.BlockSpec((BLK, 64), lambda i: (i, 0)),
        out_shape=jax.ShapeDtypeStruct((NP, 64), jnp.float32),
    )(x, nsum, W1, degp)


def _mid_body(acc_ref, yp_ref, degp_ref, b1_ref, x1_ref, u1_ref):
    a = acc_ref[...]
    di = _dinv(degp_ref[...])
    z1 = (a[0] + a[1] + yp_ref[...]) * di[:, None]
    x1 = z1 + b1_ref[...]
    x1_ref[...] = x1
    u1_ref[...] = x1 * di[:, None]


def _tc_mid(acc1, yp, degp, b1r):
    return pl.pallas_call(
        _mid_body,
        grid=(GRID,),
        in_specs=[
            pl.BlockSpec((NCORE, BLK, 64), lambda i: (0, i, 0)),
            pl.BlockSpec((BLK, 64), lambda i: (i, 0)),
            pl.BlockSpec((NCORE, BLK, 16), lambda i: (0, i, 0)),
            pl.BlockSpec((1, 64), lambda i: (0, 0)),
        ],
        out_specs=[
            pl.BlockSpec((BLK, 64), lambda i: (i, 0)),
            pl.BlockSpec((BLK, 64), lambda i: (i, 0)),
        ],
        out_shape=[
            jax.ShapeDtypeStruct((NP, 64), jnp.float32),
            jax.ShapeDtypeStruct((NP, 64), jnp.float32),
        ],
    )(acc1, yp, degp, b1r)


def _tail_body(acc_ref, u1_ref, degp_ref, x1_ref, wlt_ref, w2_ref, wlb_ref,
               blr_ref, b2r_ref, w3_ref, b3r_ref, w4_ref, b4r_ref,
               hp_ref, ch_ref):
    a = acc_ref[...]
    di = _dinv(degp_ref[...])
    z2 = (a[0] + a[1] + u1_ref[...]) * di[:, None]
    wlb = wlb_ref[...]
    wc = jnp.dot(w2_ref[...], wlb, preferred_element_type=jnp.float32)
    bc = blr_ref[...] + jnp.dot(b2r_ref[...], wlb,
                                preferred_element_type=jnp.float32)
    hp = (jnp.dot(x1_ref[...], wlt_ref[...], preferred_element_type=jnp.float32)
          + jnp.dot(z2, wc, preferred_element_type=jnp.float32) + bc)
    hp_ref[...] = hp
    t = jnp.maximum(jnp.dot(hp, w3_ref[...],
                            preferred_element_type=jnp.float32) + b3r_ref[...], 0.0)
    ch_ref[...] = jnp.dot(t, w4_ref[...],
                          preferred_element_type=jnp.float32) + b4r_ref[...]


def _tc_tail(acc2, u1, degp, x1, Wl_top, W2, Wl_bot, blr, b2r, W3, b3r, W4, b4r):
    return pl.pallas_call(
        _tail_body,
        grid=(GRID,),
        in_specs=[
            pl.BlockSpec((NCORE, BLK, 64), lambda i: (0, i, 0)),
            pl.BlockSpec((BLK, 64), lambda i: (i, 0)),
            pl.BlockSpec((NCORE, BLK, 16), lambda i: (0, i, 0)),
            pl.BlockSpec((BLK, 64), lambda i: (i, 0)),
            pl.BlockSpec((64, 128), lambda i: (0, 0)),
            pl.BlockSpec((64, 128), lambda i: (0, 0)),
            pl.BlockSpec((128, 128), lambda i: (0, 0)),
            pl.BlockSpec((1, 128), lambda i: (0, 0)),
            pl.BlockSpec((1, 128), lambda i: (0, 0)),
            pl.BlockSpec((128, 256), lambda i: (0, 0)),
            pl.BlockSpec((1, 256), lambda i: (0, 0)),
            pl.BlockSpec((256, 128), lambda i: (0, 0)),
            pl.BlockSpec((1, 128), lambda i: (0, 0)),
        ],
        out_specs=[
            pl.BlockSpec((BLK, 128), lambda i: (i, 0)),
            pl.BlockSpec((BLK, 128), lambda i: (i, 0)),
        ],
        out_shape=[
            jax.ShapeDtypeStruct((N, 128), jnp.float32),
            jax.ShapeDtypeStruct((N, 128), jnp.float32),
        ],
    )(acc2, u1, degp, x1, Wl_top, W2, Wl_bot, blr, b2r, W3, b3r, W4, b4r)


def _tf2x32(k0, k1, x0, x1):
    ks0 = _np.uint32(k0); ks1 = _np.uint32(k1)
    ks2 = _np.uint32(_np.uint32(0x1BD11BDA) ^ ks0 ^ ks1)
    ks = (ks0, ks1, ks2)
    x0 = (x0 + ks0).astype(_np.uint32); x1 = (x1 + ks1).astype(_np.uint32)
    rots = ((13, 15, 26, 6), (17, 29, 16, 24))
    for i in range(5):
        for r in rots[i % 2]:
            x0 = (x0 + x1).astype(_np.uint32)
            x1 = ((x1 << _np.uint32(r)) | (x1 >> _np.uint32(32 - r))).astype(_np.uint32)
            x1 = (x1 ^ x0).astype(_np.uint32)
        x0 = (x0 + ks[(i + 1) % 3]).astype(_np.uint32)
        x1 = (x1 + ks[(i + 2) % 3] + _np.uint32(i + 1)).astype(_np.uint32)
    return x0, x1


def _ndtri(p):
    a = [-3.969683028665376e+01, 2.209460984245205e+02, -2.759285104469687e+02,
         1.383577518672690e+02, -3.066479806614716e+01, 2.506628277459239e+00]
    b = [-5.447609879822406e+01, 1.615858368580409e+02, -1.556989798598866e+02,
         6.680131188771972e+01, -1.328068155288572e+01]
    c = [-7.784894002430293e-03, -3.223964580411365e-01, -2.400758277161838e+00,
         -2.549732539343734e+00, 4.374664141464968e+00, 2.938163982698783e+00]
    d = [7.784695709041462e-03, 3.224671290700398e-01, 2.445134137142996e+00,
         3.754408661907416e+00]
    p = _np.asarray(p, _np.float64)
    out = _np.empty_like(p)
    plow = 0.02425
    lo = p < plow; hi = p > 1 - plow; mid = ~(lo | hi)
    q = _np.sqrt(-2 * _np.log(p[lo]))
    out[lo] = (((((c[0]*q+c[1])*q+c[2])*q+c[3])*q+c[4])*q+c[5]) / ((((d[0]*q+d[1])*q+d[2])*q+d[3])*q+1)
    q = _np.sqrt(-2 * _np.log(1 - p[hi]))
    out[hi] = -(((((c[0]*q+c[1])*q+c[2])*q+c[3])*q+c[4])*q+c[5]) / ((((d[0]*q+d[1])*q+d[2])*q+d[3])*q+1)
    q = p[mid] - 0.5; r = q * q
    out[mid] = (((((a[0]*r+a[1])*r+a[2])*r+a[3])*r+a[4])*r+a[5])*q / (((((b[0]*r+b[1])*r+b[2])*r+b[3])*r+b[4])*r+1)
    return out


def _np_normal(k0, k1, n):
    cnt = _np.arange(n, dtype=_np.uint32)
    a, b = _tf2x32(k0, k1, _np.zeros(n, _np.uint32), cnt)
    bits = a ^ b
    f = ((bits >> _np.uint32(9)) | _np.uint32(0x3F800000)).view(_np.float32)
    u01 = (f - _np.float32(1.0)).astype(_np.float32)
    lo = _np.float32(_np.nextafter(_np.float32(-1.0), _np.float32(0.0)))
    hi = _np.float32(1.0)
    u = _np.maximum(lo, (u01 * (hi - lo) + lo).astype(_np.float32))
    return _ndtri((u.astype(_np.float64) + 1.0) / 2.0).astype(_np.float32)


def _const_noise():
    def fold(d):
        a, b = _tf2x32(_np.uint32(0), _np.uint32(42),
                       _np.uint32([0]), _np.uint32([d]))
        return a[0], b[0]
    k1 = fold(1)
    k2 = fold(2)
    n1 = _np_normal(k1[0], k1[1], N * 128).reshape(N, 128)
    n2 = _np_normal(k2[0], k2[1], N * 128).reshape(N, 128)
    return ((n1 + n2) * _np.float32(0.1)).astype(_np.float32)


_np = __import__("numpy")
_NSUM = _const_noise()
_ZEROS = _np.zeros((NP, 64), "float32")
_ZEROS16 = _np.zeros((NP, 16), "float32")
_ONES16 = _np.ones((CHUNK, 16), "float32")


def kernel(x, edge_index, W1, b1, W2, b2, Wl, bl, W3, b3, W4, b4):
    nsum = jnp.asarray(_NSUM)

    src3 = edge_index[0].reshape(NW, NCHUNK, CHUNK)
    dst3 = edge_index[1].reshape(NW, NCHUNK, CHUNK)
    zeros = jnp.asarray(_ZEROS)
    zeros16 = jnp.asarray(_ZEROS16)
    ones16 = jnp.asarray(_ONES16)

    degp = _sc_deg(dst3, zeros16, ones16)
    yp = _tc_head(x, nsum, W1, degp)
    acc1 = _sc_spmm(yp, src3, dst3, zeros)
    x1, u1 = _tc_mid(acc1, yp, degp, b1.reshape(1, 64))
    acc2 = _sc_spmm(u1, src3, dst3, zeros)
    hp, ch = _tc_tail(acc2, u1, degp, x1, Wl[:64], W2, Wl[64:],
                      bl.reshape(1, 128), b2.reshape(1, 128),
                      W3, b3.reshape(1, 256), W4, b4.reshape(1, 128))
    h = hp[None]
    c = ch[None]
    return (h, h, c, c)
